# baseline (speedup 1.0000x reference)
import functools

import jax
import jax.numpy as jnp
from jax import lax
from jax.experimental import pallas as pl
from jax.experimental.pallas import tpu as pltpu
from jax.experimental.pallas import tpu_sc as plsc

L_TOTAL = 1048576
NUM_VALUES = 21
NC = 2
NS = 16
NW = NC * NS
LANES = 16
CHUNK_P = 1024
NG = CHUNK_P // LANES
NBUF = 2

SC_L = 655360
CH = SC_L // NW
NCHUNK = CH // CHUNK_P

TC_L = L_TOTAL - SC_L
TC_BC = 8192
TC_NB = TC_L // TC_BC
SEQ_NB = L_TOTAL // TC_BC


def _tile_body(seq_hbm, table_hbm, out_i_hbm, tbufs, sbufs, tsems, ssems):
    wid = lax.axis_index("s") * NC + lax.axis_index("c")
    pos_base = wid * CH

    def make_copies(c, b):
        col0 = pos_base + c * CHUNK_P
        tcopy = pltpu.make_async_copy(
            table_hbm.at[:, pl.ds(col0, CHUNK_P)], tbufs[b], tsems[b])
        scopy = pltpu.make_async_copy(
            seq_hbm.at[pl.ds(col0, CHUNK_P)], sbufs[b], ssems[b])
        return tcopy, scopy

    for b in range(NBUF):
        for cp in make_copies(b, b):
            cp.start()

    lane = lax.iota(jnp.int32, LANES)

    def outer(k, acc):
        for b in range(NBUF):
            c = k * NBUF + b
            for cp in make_copies(c, b):
                cp.wait()
            tb = tbufs[b]
            sb = sbufs[b]

            def group(g, a, tb=tb, sb=sb):
                s = sb[pl.ds(g * LANES, LANES)]
                c_local = g * LANES + lane
                return a + plsc.load_gather(tb, [s, c_local])

            acc = lax.fori_loop(0, NG, group, acc, unroll=8)

            @pl.when(c + NBUF < NCHUNK)
            def _refill(c=c, b=b):
                for cp in make_copies(c + NBUF, b):
                    cp.start()
        return acc

    acc = lax.fori_loop(0, NCHUNK // NBUF, outer,
                        jnp.zeros((LANES,), jnp.float32))

    sbufs[0][pl.ds(0, LANES)] = lax.bitcast_convert_type(acc, jnp.int32)
    pltpu.sync_copy(sbufs[0].at[pl.ds(0, LANES)],
                    out_i_hbm.at[pl.ds(wid * LANES, LANES)])


@functools.partial(
    pl.kernel,
    out_type=jax.ShapeDtypeStruct((NW * LANES,), jnp.int32),
    mesh=plsc.VectorSubcoreMesh(core_axis_name="c", subcore_axis_name="s"),
    compiler_params=pltpu.CompilerParams(needs_layout_passes=False),
    scratch_types=[
        [pltpu.VMEM((NUM_VALUES, CHUNK_P), jnp.float32)] * NBUF,
        [pltpu.VMEM((CHUNK_P,), jnp.int32)] * NBUF,
        [pltpu.SemaphoreType.DMA] * NBUF,
        [pltpu.SemaphoreType.DMA] * NBUF,
    ],
)
def _gather_sum_sc(seq_hbm, table_hbm, out_i_hbm,
                   tbufs, sbufs, tsems, ssems):
    _tile_body(seq_hbm, table_hbm, out_i_hbm, tbufs, sbufs, tsems, ssems)


def _tc_body(seq_ref, tbl_ref, out_ref):
    j = pl.program_id(0)
    data = tbl_ref[...]
    s = seq_ref[0, 0, :]
    rows = lax.broadcasted_iota(jnp.int32, (NUM_VALUES, TC_BC), 0)
    sel = jnp.where(rows == s[None, :], data, 0.0)
    colsum = jnp.sum(sel, axis=0)
    partial = jnp.sum(colsum.reshape(TC_BC // 128, 128), axis=0)

    @pl.when(j == 0)
    def _init():
        out_ref[0, 0, :] = partial

    @pl.when(j != 0)
    def _accum():
        out_ref[0, 0, :] += partial


_tc_sum = pl.pallas_call(
    _tc_body,
    grid=(TC_NB,),
    in_specs=[
        pl.BlockSpec((1, 1, TC_BC), lambda j: (SC_L // TC_BC + j, 0, 0)),
        pl.BlockSpec((NUM_VALUES, TC_BC), lambda j: (0, SC_L // TC_BC + j)),
    ],
    out_specs=pl.BlockSpec((1, 1, 128), lambda j: (0, 0, 0)),
    out_shape=jax.ShapeDtypeStruct((1, 1, 128), jnp.float32),
)


def kernel(sequence, site_probabilities):
    table_t = site_probabilities.T
    seq = sequence.astype(jnp.int32)
    tc_partials = _tc_sum(seq.reshape(SEQ_NB, 1, TC_BC), table_t)
    sc_partials = _gather_sum_sc(seq, table_t)
    total = (jnp.sum(lax.bitcast_convert_type(sc_partials, jnp.float32))
             + jnp.sum(tc_partials))
    return -total


# speedup vs baseline: 1.0790x; 1.0790x over previous
import functools

import jax
import jax.numpy as jnp
from jax import lax
from jax.experimental import pallas as pl
from jax.experimental.pallas import tpu as pltpu
from jax.experimental.pallas import tpu_sc as plsc

L_TOTAL = 1048576
NUM_VALUES = 21
NC = 2
NS = 16
NW = NC * NS
LANES = 16
CHUNK_P = 1024
NG = CHUNK_P // LANES
NBUF = 2

SC_L = 720896
CH = SC_L // NW
NCHUNK = CH // CHUNK_P

TC_L = L_TOTAL - SC_L
TC_BC = 8192
TC_NB = TC_L // TC_BC
SEQ_NB = L_TOTAL // TC_BC


def _tile_body(seq_hbm, table_hbm, out_i_hbm, tbufs, sbufs, tsems, ssems):
    wid = lax.axis_index("s") * NC + lax.axis_index("c")
    pos_base = wid * CH

    def make_copies(c, b):
        col0 = pos_base + c * CHUNK_P
        tcopy = pltpu.make_async_copy(
            table_hbm.at[:, pl.ds(col0, CHUNK_P)], tbufs[b], tsems[b])
        scopy = pltpu.make_async_copy(
            seq_hbm.at[pl.ds(col0, CHUNK_P)], sbufs[b], ssems[b])
        return tcopy, scopy

    for b in range(NBUF):
        for cp in make_copies(b, b):
            cp.start()

    lane = lax.iota(jnp.int32, LANES)

    def outer(k, acc):
        for b in range(NBUF):
            c = k * NBUF + b
            for cp in make_copies(c, b):
                cp.wait()
            tb = tbufs[b]
            sb = sbufs[b]

            def group(g, a, tb=tb, sb=sb):
                s = sb[pl.ds(g * LANES, LANES)]
                c_local = g * LANES + lane
                return a + plsc.load_gather(tb, [s, c_local])

            acc = lax.fori_loop(0, NG, group, acc, unroll=8)

            @pl.when(c + NBUF < NCHUNK)
            def _refill(c=c, b=b):
                for cp in make_copies(c + NBUF, b):
                    cp.start()
        return acc

    acc = lax.fori_loop(0, NCHUNK // NBUF, outer,
                        jnp.zeros((LANES,), jnp.float32))

    sbufs[0][pl.ds(0, LANES)] = lax.bitcast_convert_type(acc, jnp.int32)
    pltpu.sync_copy(sbufs[0].at[pl.ds(0, LANES)],
                    out_i_hbm.at[pl.ds(wid * LANES, LANES)])


@functools.partial(
    pl.kernel,
    out_type=jax.ShapeDtypeStruct((NW * LANES,), jnp.int32),
    mesh=plsc.VectorSubcoreMesh(core_axis_name="c", subcore_axis_name="s"),
    compiler_params=pltpu.CompilerParams(needs_layout_passes=False),
    scratch_types=[
        [pltpu.VMEM((NUM_VALUES, CHUNK_P), jnp.float32)] * NBUF,
        [pltpu.VMEM((CHUNK_P,), jnp.int32)] * NBUF,
        [pltpu.SemaphoreType.DMA] * NBUF,
        [pltpu.SemaphoreType.DMA] * NBUF,
    ],
)
def _gather_sum_sc(seq_hbm, table_hbm, out_i_hbm,
                   tbufs, sbufs, tsems, ssems):
    _tile_body(seq_hbm, table_hbm, out_i_hbm, tbufs, sbufs, tsems, ssems)


def _tc_body(seq_ref, tbl_ref, out_ref):
    j = pl.program_id(0)
    data = tbl_ref[...]
    s = seq_ref[0, 0, :]
    rows = lax.broadcasted_iota(jnp.int32, (NUM_VALUES, TC_BC), 0)
    sel = jnp.where(rows == s[None, :], data, 0.0)
    colsum = jnp.sum(sel, axis=0)
    partial = jnp.sum(colsum.reshape(TC_BC // 128, 128), axis=0)

    @pl.when(j == 0)
    def _init():
        out_ref[0, 0, :] = partial

    @pl.when(j != 0)
    def _accum():
        out_ref[0, 0, :] += partial


_tc_sum = pl.pallas_call(
    _tc_body,
    grid=(TC_NB,),
    in_specs=[
        pl.BlockSpec((1, 1, TC_BC), lambda j: (SC_L // TC_BC + j, 0, 0)),
        pl.BlockSpec((NUM_VALUES, TC_BC), lambda j: (0, SC_L // TC_BC + j)),
    ],
    out_specs=pl.BlockSpec((1, 1, 128), lambda j: (0, 0, 0)),
    out_shape=jax.ShapeDtypeStruct((1, 1, 128), jnp.float32),
)


def kernel(sequence, site_probabilities):
    table_t = site_probabilities.T
    seq = sequence.astype(jnp.int32)
    sc_partials = _gather_sum_sc(seq, table_t)
    tc_partials = _tc_sum(seq.reshape(SEQ_NB, 1, TC_BC), table_t)
    total = (jnp.sum(lax.bitcast_convert_type(sc_partials, jnp.float32))
             + jnp.sum(tc_partials))
    return -total
